# baseline (speedup 1.0000x reference)
import functools

import jax
import jax.numpy as jnp
from jax import lax
from jax.experimental import pallas as pl
from jax.experimental.pallas import tpu as pltpu
from jax.experimental.pallas import tpu_sc as plsc

N = 10000
E = 320000
D = 128
H = 128
DEPTH_ = 10

NC = 2
NS = 16
L = 16

WCH = 125
WNC = E // (NC * NS) // WCH
WHALF = WNC // 2
NPAD = 10240
WSTR = NPAD // NS
ZROWS = 128


def _wide_body(feat_hbm, src_hbm, dst_hbm, zer_hbm, y_hbm, src_v, dst_v,
               rows0_v, rows1_v, gsem0, gsem1, ssem0, ssem1, acc):
    cid = lax.axis_index("c")
    sid = lax.axis_index("s")
    rows = (rows0_v, rows1_v)
    gsems = (gsem0, gsem1)
    ssems = (ssem0, ssem1)
    for j in range(WSTR // ZROWS):
        pltpu.sync_copy(zer_hbm,
                        acc.at[pl.ds(sid * WSTR + j * ZROWS, ZROWS)])
    plsc.subcore_barrier()

    def gfire(g, b):
        pltpu.async_copy(feat_hbm.at[src_v.at[g]], rows[b], gsems[b])

    def gdrain(b):
        pltpu.make_async_copy(feat_hbm.at[src_v.at[0]], rows[b],
                              gsems[b]).wait()

    def sfire(g, b):
        pltpu.async_copy(rows[b], acc.at[dst_v.at[g]], ssems[b], add=True)

    def sdrain(b):
        pltpu.make_async_copy(rows[b], acc.at[dst_v.at[0]],
                              ssems[b]).wait()

    for h in range(2):
        pltpu.sync_copy(src_hbm.at[cid, sid, pl.ds(h * WHALF, WHALF)], src_v)
        pltpu.sync_copy(dst_hbm.at[cid, sid, pl.ds(h * WHALF, WHALF)], dst_v)
        gfire(0, 0)

        def chunk2(i, carry):
            g = i * 2
            for b in range(2):
                gdrain(b)
                sfire(g + b, b)

                @pl.when(g + b + 1 < WHALF)
                def _():
                    @pl.when(g + b >= 1)
                    def _():
                        sdrain(1 - b)

                    gfire(g + b + 1, 1 - b)
            return carry

        lax.fori_loop(0, WHALF // 2, chunk2, 0)
        sdrain(0)
        sdrain(1)
    plsc.subcore_barrier()
    pltpu.sync_copy(acc.at[pl.ds(sid * WSTR, WSTR)],
                    y_hbm.at[cid, pl.ds(sid * WSTR, WSTR)])


_wide = pl.kernel(
    _wide_body,
    out_type=jax.ShapeDtypeStruct((NC, NPAD, D), jnp.float32),
    mesh=plsc.VectorSubcoreMesh(core_axis_name="c", subcore_axis_name="s",
                                num_cores=NC, num_subcores=NS),
    scratch_types=[
        pltpu.VMEM((WHALF, WCH), jnp.int32),
        pltpu.VMEM((WHALF, WCH), jnp.int32),
        pltpu.VMEM((WCH, D), jnp.float32),
        pltpu.VMEM((WCH, D), jnp.float32),
        pltpu.SemaphoreType.DMA,
        pltpu.SemaphoreType.DMA,
        pltpu.SemaphoreType.DMA,
        pltpu.SemaphoreType.DMA,
        pltpu.VMEM_SHARED((NPAD, D), jnp.float32),
    ],
    compiler_params=pltpu.CompilerParams(needs_layout_passes=False),
)


def _dot(a, b):
    return jnp.dot(a.astype(jnp.bfloat16), b.astype(jnp.bfloat16),
                   preferred_element_type=jnp.float32)


def _tc_layer_body(y_ref, W_ref, b_ref, o_ref, *, relu):
    y = y_ref[0] + y_ref[1]
    r = _dot(y, W_ref[...]) + b_ref[...][None, :]
    if relu:
        r = jnp.maximum(r, 0.0)
    o_ref[...] = r


def _tc_final_body(y_ref, W_ref, b_ref, Wo_ref, bo_ref, o_ref):
    y = y_ref[0] + y_ref[1]
    r = _dot(y, W_ref[...]) + b_ref[...][None, :]
    o_ref[...] = _dot(r, Wo_ref[...]) + bo_ref[...][None, :]


_TC_BLK = 1024
_Y_SPEC = pl.BlockSpec((NC, _TC_BLK, D), lambda i: (0, i, 0))
_W_SPEC = pl.BlockSpec((D, H), lambda i: (0, 0))
_B_SPEC = pl.BlockSpec((H,), lambda i: (0,))


def _tc_layer(y_parts, W, b, relu):
    return pl.pallas_call(
        functools.partial(_tc_layer_body, relu=relu),
        grid=(NPAD // _TC_BLK,),
        in_specs=[_Y_SPEC, _W_SPEC, _B_SPEC],
        out_specs=pl.BlockSpec((_TC_BLK, H), lambda i: (i, 0)),
        out_shape=jax.ShapeDtypeStruct((NPAD, H), jnp.float32),
    )(y_parts, W, b)


def _tc_final(y_parts, W, b, W_out, b_out):
    return pl.pallas_call(
        _tc_final_body,
        grid=(NPAD // _TC_BLK,),
        in_specs=[
            _Y_SPEC, _W_SPEC, _B_SPEC,
            pl.BlockSpec((H, 1), lambda i: (0, 0)),
            pl.BlockSpec((1,), lambda i: (0,)),
        ],
        out_specs=pl.BlockSpec((_TC_BLK, 1), lambda i: (i, 0)),
        out_shape=jax.ShapeDtypeStruct((NPAD, 1), jnp.float32),
    )(y_parts, W, b, W_out, b_out)


def kernel(features, edge_index, W_in, b_in, Ws, bs, W_out, b_out):
    src = edge_index[0].astype(jnp.int32)
    dst = edge_index[1].astype(jnp.int32)
    srcW = src.reshape(NC, NS, WNC, WCH)
    dstW = dst.reshape(NC, NS, WNC, WCH)
    zer = jnp.zeros((ZROWS, D), jnp.float32)

    x = jnp.zeros((NPAD, D), jnp.float32).at[:N].set(features)
    x = _tc_layer(_wide(x, srcW, dstW, zer), W_in, b_in, relu=True)
    for i in range(DEPTH_ - 1):
        x = _tc_layer(_wide(x, srcW, dstW, zer), Ws[i], bs[i], relu=False)
    out = _tc_final(_wide(x, srcW, dstW, zer), Ws[DEPTH_ - 1], bs[DEPTH_ - 1],
                    W_out, b_out)
    return out[:N]


# speedup vs baseline: 1.0064x; 1.0064x over previous
import functools

import jax
import jax.numpy as jnp
from jax import lax
from jax.experimental import pallas as pl
from jax.experimental.pallas import tpu as pltpu
from jax.experimental.pallas import tpu_sc as plsc

N = 10000
E = 320000
D = 128
H = 128
DEPTH_ = 10

NC = 2
NS = 16
L = 16

WCH = 125
WNC = E // (NC * NS) // WCH
WHALF = WNC // 2
NPAD = 10240
WSTR = NPAD // NS
ZROWS = 128


def _wide_body(feat_hbm, src_hbm, dst_hbm, zer_hbm, y_hbm, src_v, dst_v,
               rows0_v, rows1_v, gsem0, gsem1, ssem0, ssem1, acc):
    cid = lax.axis_index("c")
    sid = lax.axis_index("s")
    rows = (rows0_v, rows1_v)
    gsems = (gsem0, gsem1)
    ssems = (ssem0, ssem1)
    pltpu.sync_copy(src_hbm.at[cid, sid, pl.ds(0, WHALF)], src_v)
    pltpu.sync_copy(dst_hbm.at[cid, sid, pl.ds(0, WHALF)], dst_v)
    pltpu.async_copy(feat_hbm.at[src_v.at[0]], rows0_v, gsem0)
    for j in range(WSTR // ZROWS):
        pltpu.async_copy(zer_hbm,
                         acc.at[pl.ds(sid * WSTR + j * ZROWS, ZROWS)], ssem0)
    for j in range(WSTR // ZROWS):
        pltpu.make_async_copy(zer_hbm,
                              acc.at[pl.ds(0, ZROWS)], ssem0).wait()
    plsc.subcore_barrier()

    def gfire(g, b):
        pltpu.async_copy(feat_hbm.at[src_v.at[g]], rows[b], gsems[b])

    def gdrain(b):
        pltpu.make_async_copy(feat_hbm.at[src_v.at[0]], rows[b],
                              gsems[b]).wait()

    def sfire(g, b):
        pltpu.async_copy(rows[b], acc.at[dst_v.at[g]], ssems[b], add=True)

    def sdrain(b):
        pltpu.make_async_copy(rows[b], acc.at[dst_v.at[0]],
                              ssems[b]).wait()

    for h in range(2):
        if h > 0:
            pltpu.sync_copy(src_hbm.at[cid, sid, pl.ds(h * WHALF, WHALF)],
                            src_v)
            pltpu.sync_copy(dst_hbm.at[cid, sid, pl.ds(h * WHALF, WHALF)],
                            dst_v)
            gfire(0, 0)

        def chunk2(i, carry):
            g = i * 2
            for b in range(2):
                gdrain(b)
                sfire(g + b, b)

                @pl.when(g + b + 1 < WHALF)
                def _():
                    @pl.when(g + b >= 1)
                    def _():
                        sdrain(1 - b)

                    gfire(g + b + 1, 1 - b)
            return carry

        lax.fori_loop(0, WHALF // 2, chunk2, 0)
        sdrain(0)
        sdrain(1)
    plsc.subcore_barrier()
    pltpu.sync_copy(acc.at[pl.ds(sid * WSTR, WSTR)],
                    y_hbm.at[cid, pl.ds(sid * WSTR, WSTR)])


_wide = pl.kernel(
    _wide_body,
    out_type=jax.ShapeDtypeStruct((NC, NPAD, D), jnp.float32),
    mesh=plsc.VectorSubcoreMesh(core_axis_name="c", subcore_axis_name="s",
                                num_cores=NC, num_subcores=NS),
    scratch_types=[
        pltpu.VMEM((WHALF, WCH), jnp.int32),
        pltpu.VMEM((WHALF, WCH), jnp.int32),
        pltpu.VMEM((WCH, D), jnp.float32),
        pltpu.VMEM((WCH, D), jnp.float32),
        pltpu.SemaphoreType.DMA,
        pltpu.SemaphoreType.DMA,
        pltpu.SemaphoreType.DMA,
        pltpu.SemaphoreType.DMA,
        pltpu.VMEM_SHARED((NPAD, D), jnp.float32),
    ],
    compiler_params=pltpu.CompilerParams(needs_layout_passes=False),
)


def _dot(a, b):
    return jnp.dot(a.astype(jnp.bfloat16), b.astype(jnp.bfloat16),
                   preferred_element_type=jnp.float32)


def _tc_layer_body(y_ref, W_ref, b_ref, o_ref, *, relu):
    y = y_ref[0] + y_ref[1]
    r = _dot(y, W_ref[...]) + b_ref[...][None, :]
    if relu:
        r = jnp.maximum(r, 0.0)
    o_ref[...] = r


def _tc_final_body(y_ref, W_ref, b_ref, Wo_ref, bo_ref, o_ref):
    y = y_ref[0] + y_ref[1]
    r = _dot(y, W_ref[...]) + b_ref[...][None, :]
    o_ref[...] = _dot(r, Wo_ref[...]) + bo_ref[...][None, :]


_TC_BLK = 1024
_Y_SPEC = pl.BlockSpec((NC, _TC_BLK, D), lambda i: (0, i, 0))
_W_SPEC = pl.BlockSpec((D, H), lambda i: (0, 0))
_B_SPEC = pl.BlockSpec((H,), lambda i: (0,))


def _tc_layer(y_parts, W, b, relu):
    return pl.pallas_call(
        functools.partial(_tc_layer_body, relu=relu),
        grid=(NPAD // _TC_BLK,),
        in_specs=[_Y_SPEC, _W_SPEC, _B_SPEC],
        out_specs=pl.BlockSpec((_TC_BLK, H), lambda i: (i, 0)),
        out_shape=jax.ShapeDtypeStruct((NPAD, H), jnp.float32),
    )(y_parts, W, b)


def _tc_final(y_parts, W, b, W_out, b_out):
    return pl.pallas_call(
        _tc_final_body,
        grid=(NPAD // _TC_BLK,),
        in_specs=[
            _Y_SPEC, _W_SPEC, _B_SPEC,
            pl.BlockSpec((H, 1), lambda i: (0, 0)),
            pl.BlockSpec((1,), lambda i: (0,)),
        ],
        out_specs=pl.BlockSpec((_TC_BLK, 1), lambda i: (i, 0)),
        out_shape=jax.ShapeDtypeStruct((NPAD, 1), jnp.float32),
    )(y_parts, W, b, W_out, b_out)


def kernel(features, edge_index, W_in, b_in, Ws, bs, W_out, b_out):
    src = edge_index[0].astype(jnp.int32)
    dst = edge_index[1].astype(jnp.int32)
    srcW = src.reshape(NC, NS, WNC, WCH)
    dstW = dst.reshape(NC, NS, WNC, WCH)
    zer = jnp.zeros((ZROWS, D), jnp.float32)

    x = jnp.zeros((NPAD, D), jnp.float32).at[:N].set(features)
    x = _tc_layer(_wide(x, srcW, dstW, zer), W_in, b_in, relu=True)
    for i in range(DEPTH_ - 1):
        x = _tc_layer(_wide(x, srcW, dstW, zer), Ws[i], bs[i], relu=False)
    out = _tc_final(_wide(x, srcW, dstW, zer), Ws[DEPTH_ - 1], bs[DEPTH_ - 1],
                    W_out, b_out)
    return out[:N]


# speedup vs baseline: 1.0090x; 1.0026x over previous
import functools

import jax
import jax.numpy as jnp
from jax import lax
from jax.experimental import pallas as pl
from jax.experimental.pallas import tpu as pltpu
from jax.experimental.pallas import tpu_sc as plsc

N = 10000
E = 320000
D = 128
H = 128
DEPTH_ = 10

NC = 2
NS = 16

WCH = 125
WNC = E // (NC * NS) // WCH
WHALF = WNC // 2
NPAD = 10240
WSTR = NPAD // NS
ZROWS = 128


def _wide_body(feat_hbm, src_hbm, dst_hbm, zer_hbm, y_hbm, src_v, dst_v,
               rows0_v, rows1_v, gsem0, gsem1, ssem0, ssem1, acc):
    cid = lax.axis_index("c")
    sid = lax.axis_index("s")
    rows = (rows0_v, rows1_v)
    gsems = (gsem0, gsem1)
    ssems = (ssem0, ssem1)
    pltpu.sync_copy(src_hbm.at[cid, sid, pl.ds(0, WHALF)], src_v)
    pltpu.sync_copy(dst_hbm.at[cid, sid, pl.ds(0, WHALF)], dst_v)
    pltpu.async_copy(feat_hbm.at[src_v.at[0]], rows0_v, gsem0)
    for j in range(WSTR // ZROWS):
        pltpu.async_copy(zer_hbm,
                         acc.at[pl.ds(sid * WSTR + j * ZROWS, ZROWS)], ssem0)
    for j in range(WSTR // ZROWS):
        pltpu.make_async_copy(zer_hbm,
                              acc.at[pl.ds(0, ZROWS)], ssem0).wait()
    plsc.subcore_barrier()

    def gfire(g, b):
        pltpu.async_copy(feat_hbm.at[src_v.at[g]], rows[b], gsems[b])

    def gdrain(b):
        pltpu.make_async_copy(feat_hbm.at[src_v.at[0]], rows[b],
                              gsems[b]).wait()

    def sfire(g, b):
        pltpu.async_copy(rows[b], acc.at[dst_v.at[g]], ssems[b], add=True)

    def sdrain(b):
        pltpu.make_async_copy(rows[b], acc.at[dst_v.at[0]],
                              ssems[b]).wait()

    for h in range(2):
        if h > 0:
            pltpu.sync_copy(src_hbm.at[cid, sid, pl.ds(h * WHALF, WHALF)],
                            src_v)
            pltpu.sync_copy(dst_hbm.at[cid, sid, pl.ds(h * WHALF, WHALF)],
                            dst_v)
            gfire(0, 0)

        def chunk2(i, carry):
            g = i * 2
            for b in range(2):
                gdrain(b)
                sfire(g + b, b)

                @pl.when(g + b + 1 < WHALF)
                def _():
                    @pl.when(g + b >= 1)
                    def _():
                        sdrain(1 - b)

                    gfire(g + b + 1, 1 - b)
            return carry

        lax.fori_loop(0, WHALF // 2, chunk2, 0)
        sdrain(0)
        sdrain(1)
    plsc.subcore_barrier()
    pltpu.sync_copy(acc.at[pl.ds(sid * WSTR, WSTR)],
                    y_hbm.at[cid, pl.ds(sid * WSTR, WSTR)])


_wide = pl.kernel(
    _wide_body,
    out_type=jax.ShapeDtypeStruct((NC, NPAD, D), jnp.float32),
    mesh=plsc.VectorSubcoreMesh(core_axis_name="c", subcore_axis_name="s",
                                num_cores=NC, num_subcores=NS),
    scratch_types=[
        pltpu.VMEM((WHALF, WCH), jnp.int32),
        pltpu.VMEM((WHALF, WCH), jnp.int32),
        pltpu.VMEM((WCH, D), jnp.float32),
        pltpu.VMEM((WCH, D), jnp.float32),
        pltpu.SemaphoreType.DMA,
        pltpu.SemaphoreType.DMA,
        pltpu.SemaphoreType.DMA,
        pltpu.SemaphoreType.DMA,
        pltpu.VMEM_SHARED((NPAD, D), jnp.float32),
    ],
    compiler_params=pltpu.CompilerParams(needs_layout_passes=False),
)


def _dot(a, b):
    return jnp.dot(a.astype(jnp.bfloat16), b.astype(jnp.bfloat16),
                   preferred_element_type=jnp.float32)


def _tc_layer_body(y_ref, W_ref, b_ref, o_ref, *, relu):
    y = y_ref[0] + y_ref[1]
    r = _dot(y, W_ref[...]) + b_ref[...][None, :]
    if relu:
        r = jnp.maximum(r, 0.0)
    o_ref[...] = r


def _tc_final_body(y_ref, W_ref, b_ref, Wo_ref, bo_ref, o_ref):
    y = y_ref[0] + y_ref[1]
    r = _dot(y, W_ref[...]) + b_ref[...][None, :]
    o_ref[...] = _dot(r, Wo_ref[...]) + bo_ref[...][None, :]


_TC_BLK = 1024
_Y_SPEC = pl.BlockSpec((NC, _TC_BLK, D), lambda i: (0, i, 0))
_W_SPEC = pl.BlockSpec((D, H), lambda i: (0, 0))
_B_SPEC = pl.BlockSpec((H,), lambda i: (0,))


def _tc_layer(y_parts, W, b, relu):
    return pl.pallas_call(
        functools.partial(_tc_layer_body, relu=relu),
        grid=(NPAD // _TC_BLK,),
        in_specs=[_Y_SPEC, _W_SPEC, _B_SPEC],
        out_specs=pl.BlockSpec((_TC_BLK, H), lambda i: (i, 0)),
        out_shape=jax.ShapeDtypeStruct((NPAD, H), jnp.float32),
    )(y_parts, W, b)


def _tc_final(y_parts, W, b, W_out, b_out):
    return pl.pallas_call(
        _tc_final_body,
        grid=(NPAD // _TC_BLK,),
        in_specs=[
            _Y_SPEC, _W_SPEC, _B_SPEC,
            pl.BlockSpec((H, 1), lambda i: (0, 0)),
            pl.BlockSpec((1,), lambda i: (0,)),
        ],
        out_specs=pl.BlockSpec((_TC_BLK, 1), lambda i: (i, 0)),
        out_shape=jax.ShapeDtypeStruct((NPAD, 1), jnp.float32),
    )(y_parts, W, b, W_out, b_out)


def kernel(features, edge_index, W_in, b_in, Ws, bs, W_out, b_out):
    src = edge_index[0].astype(jnp.int32)
    dst = edge_index[1].astype(jnp.int32)
    srcW = src.reshape(NC, NS, WNC, WCH)
    dstW = dst.reshape(NC, NS, WNC, WCH)
    zer = jnp.zeros((ZROWS, D), jnp.float32)

    x = jnp.zeros((NPAD, D), jnp.float32).at[:N].set(features)
    x = _tc_layer(_wide(x, srcW, dstW, zer), W_in, b_in, relu=True)
    for i in range(DEPTH_ - 1):
        x = _tc_layer(_wide(x, srcW, dstW, zer), Ws[i], bs[i], relu=False)
    out = _tc_final(_wide(x, srcW, dstW, zer), Ws[DEPTH_ - 1], bs[DEPTH_ - 1],
                    W_out, b_out)
    return out[:N]
